# baseline (speedup 1.0000x reference)
import functools

import jax
import jax.numpy as jnp
from jax import lax
from jax.experimental import pallas as pl
from jax.experimental.pallas import tpu as pltpu
from jax.experimental.pallas import tpu_sc as plsc

N = 10000
E = 320000
D = 128
C = 40

NC = 2
NS = 16
NW = NC * NS

BLK = 64
EPW_BLKS = 160
HOLD = 40
NBUF = 4
E_PAD = NW * BLK * EPW_BLKS
TOT_BLKS = E_PAD // BLK
BLKS_C0 = 320
BLKS_C1 = 0
NROW = 10112
STRIPE = NROW // NS
DW = 128
ZCH = 128
DBLK = 128
DEG_BLKS = E_PAD // (NW * DBLK)

_mesh = plsc.VectorSubcoreMesh(
    core_axis_name="c", subcore_axis_name="s", num_cores=NC, num_subcores=NS)


def _zero_vmem_rows(ref, nrows, width):
  def row(r, _):
    for j in range(width // 16):
      ref[r, pl.ds(j * 16, 16)] = jnp.zeros((16,), jnp.float32)
    return 0
  lax.fori_loop(0, nrows, row, 0)


def _stripe_copy_zero(zbuf, zrows, acc, base, total):
  off = 0
  while off < total:
    ch = min(zrows, total - off)
    pltpu.sync_copy(zbuf.at[pl.ds(0, ch)], acc.at[pl.ds(base + off, ch)])
    off += ch


@functools.partial(
    pl.kernel,
    out_type=jax.ShapeDtypeStruct((NC, NROW, DW), jnp.float32),
    mesh=_mesh,
    scratch_types=[
        pltpu.VMEM((DEG_BLKS, DBLK), jnp.int32),
        pltpu.VMEM((ZCH, DW), jnp.float32),
        pltpu.VMEM((ZCH, DW), jnp.float32),
        pltpu.VMEM_SHARED((NROW, DW), jnp.float32),
    ],
)
def _sc_deg(dst_hbm, out_hbm, idst, ones_v, zbuf, acc):
  cid = lax.axis_index("c")
  sid = lax.axis_index("s")
  wid = sid * NC + cid

  def fill(r, _):
    for j in range(DW // 16):
      ones_v[r, pl.ds(j * 16, 16)] = jnp.ones((16,), jnp.float32)
      zbuf[r, pl.ds(j * 16, 16)] = jnp.zeros((16,), jnp.float32)
    return 0
  lax.fori_loop(0, ZCH, fill, 0)
  _stripe_copy_zero(zbuf, ZCH, acc, sid * STRIPE, STRIPE)
  pltpu.sync_copy(dst_hbm.at[wid], idst)
  plsc.subcore_barrier()

  def rnd(g, _):
    pltpu.sync_copy(ones_v.at[pl.ds(0, DBLK)], acc.at[idst.at[g]], add=True)
    return 0
  lax.fori_loop(0, DEG_BLKS, rnd, 0)
  plsc.subcore_barrier()
  pltpu.sync_copy(acc.at[pl.ds(sid * STRIPE, STRIPE)],
                  out_hbm.at[cid, pl.ds(sid * STRIPE, STRIPE)])


@functools.partial(
    pl.kernel,
    out_type=jax.ShapeDtypeStruct((NC, NROW, D), jnp.float32),
    mesh=_mesh,
    scratch_types=[
        pltpu.VMEM((HOLD, BLK), jnp.int32),
        pltpu.VMEM((HOLD, BLK), jnp.int32),
        [pltpu.VMEM((BLK, D), jnp.float32)] * NBUF,
        pltpu.VMEM_SHARED((NROW, D), jnp.float32),
        [pltpu.SemaphoreType.DMA] * NBUF,
    ],
)
def _sc_agg(hp_hbm, src_hbm, dst_hbm, out_hbm, isrc, idst, rows, acc, sems):
  cid = lax.axis_index("c")
  sid = lax.axis_index("s")

  _zero_vmem_rows(rows[0], BLK, D)
  _stripe_copy_zero(rows[0], BLK, acc, sid * STRIPE, STRIPE)
  plsc.subcore_barrier()

  base = jnp.where(cid == 0, sid * BLKS_C0, NS * BLKS_C0 + sid * BLKS_C1)
  nchunks = jnp.where(cid == 0, BLKS_C0 // HOLD, BLKS_C1 // HOLD)

  def chunk(q, _):
    b0 = base + q * HOLD
    pltpu.sync_copy(src_hbm.at[pl.ds(b0, HOLD)], isrc)
    pltpu.sync_copy(dst_hbm.at[pl.ds(b0, HOLD)], idst)
    for j in range(NBUF):
      pltpu.async_copy(hp_hbm.at[isrc.at[j]], rows[j], sems[j])

    def rnd(g, _):
      for j in range(NBUF):
        b = NBUF * g + j
        pltpu.make_async_copy(hp_hbm.at[isrc.at[b]], rows[j], sems[j]).wait()
        pltpu.sync_copy(rows[j], acc.at[idst.at[b]], add=True)

        @pl.when(b + NBUF < HOLD)
        def _():
          pltpu.async_copy(hp_hbm.at[isrc.at[b + NBUF]], rows[j], sems[j])
      return 0

    lax.fori_loop(0, HOLD // NBUF, rnd, 0)
    return 0

  lax.fori_loop(0, nchunks, chunk, 0)
  plsc.subcore_barrier()
  pltpu.sync_copy(acc.at[pl.ds(sid * STRIPE, STRIPE)],
                  out_hbm.at[cid, pl.ds(sid * STRIPE, STRIPE)])


_RB = 1000


def _dinv_body(d0_ref, d1_ref, o_ref):
  deg = d0_ref[:, 0:1] + d1_ref[:, 0:1] + 1.0
  o_ref[...] = jnp.broadcast_to(lax.rsqrt(deg), o_ref.shape)


def _tc_dinv(deg_parts):
  return pl.pallas_call(
      _dinv_body,
      out_shape=jax.ShapeDtypeStruct((NROW, D), jnp.float32),
  )(deg_parts[0], deg_parts[1])


def _mm_scale_body(x_ref, w_ref, dinv_ref, o_ref):
  h = jnp.dot(x_ref[...], w_ref[...], preferred_element_type=jnp.float32)
  o_ref[...] = h * dinv_ref[...]


def _tc_mm_scale(x, w, dinv):
  grid = (N // _RB,)
  return pl.pallas_call(
      _mm_scale_body,
      grid=grid,
      in_specs=[
          pl.BlockSpec((_RB, D), lambda i: (i, 0)),
          pl.BlockSpec((D, D), lambda i: (0, 0)),
          pl.BlockSpec((_RB, D), lambda i: (i, 0)),
      ],
      out_specs=pl.BlockSpec((_RB, D), lambda i: (i, 0)),
      out_shape=jax.ShapeDtypeStruct((N, D), jnp.float32),
  )(x, w, dinv)


def _combine_mm_body(p0_ref, p1_ref, hp_ref, dinv_ref, b_ref, w_ref, o_ref):
  a = dinv_ref[...] * (p0_ref[...] + p1_ref[...] + hp_ref[...]) + b_ref[...]
  a = jnp.maximum(a, 0.0)
  o_ref[...] = jnp.dot(a, w_ref[...], preferred_element_type=jnp.float32)


def _scale_after_body(p0_ref, p1_ref, hp_ref, dinv_ref, b_ref, w_ref, o_ref):
  _combine_mm_body(p0_ref, p1_ref, hp_ref, dinv_ref, b_ref, w_ref, o_ref)
  o_ref[...] = o_ref[...] * dinv_ref[...]


def _final_body(p0_ref, p1_ref, hp_ref, dinv_ref, b_ref, w_ref, bf_ref, o_ref):
  a = dinv_ref[...] * (p0_ref[...] + p1_ref[...] + hp_ref[...]) + b_ref[...]
  a = jnp.maximum(a, 0.0)
  o_ref[...] = jnp.dot(a, w_ref[...], preferred_element_type=jnp.float32) + bf_ref[...]


def _tc_combine(body, parts, hp, dinv, b_row, w, extra=()):
  grid = (N // _RB,)
  blk = pl.BlockSpec((_RB, D), lambda i: (i, 0))
  full = pl.BlockSpec((D, D), lambda i: (0, 0))
  brow = pl.BlockSpec((1, D), lambda i: (0, 0))
  in_specs = [blk, blk, blk, blk, brow, full] + [brow] * len(extra)
  return pl.pallas_call(
      body,
      grid=grid,
      in_specs=in_specs,
      out_specs=blk,
      out_shape=jax.ShapeDtypeStruct((N, D), jnp.float32),
  )(parts[0], parts[1], hp, dinv, b_row, w, *extra)


def kernel(x, edge_index, W1, b1, Wh0, bh0, Wh1, bh1, Wf, bf):
  src = edge_index[0]
  dst = edge_index[1]
  pad = E_PAD - E
  src_p = jnp.concatenate([src, jnp.zeros((pad,), jnp.int32)])
  dst_p = jnp.concatenate([dst, jnp.full((pad,), N, jnp.int32)])
  src_r = src_p.reshape(TOT_BLKS, BLK)
  dst_r = dst_p.reshape(TOT_BLKS, BLK)

  deg_parts = _sc_deg(dst_p.reshape(NW, DEG_BLKS, DBLK))
  dinv_full = _tc_dinv(deg_parts)
  dinv = dinv_full[:N]

  wf_pad = jnp.zeros((D, D), jnp.float32).at[:, :C].set(Wf)
  bf_pad = jnp.zeros((1, D), jnp.float32).at[0, :C].set(bf)

  hp = _tc_mm_scale(x, W1, dinv)
  parts = _sc_agg(hp, src_r, dst_r)
  hp = _tc_combine(_scale_after_body, (parts[0][:N], parts[1][:N]), hp, dinv,
                   b1.reshape(1, D), Wh0)
  parts = _sc_agg(hp, src_r, dst_r)
  hp = _tc_combine(_scale_after_body, (parts[0][:N], parts[1][:N]), hp, dinv,
                   bh0.reshape(1, D), Wh1)
  parts = _sc_agg(hp, src_r, dst_r)
  out = _tc_combine(_final_body, (parts[0][:N], parts[1][:N]), hp, dinv,
                    bh1.reshape(1, D), wf_pad, extra=(bf_pad,))
  return out[:, :C]


# speedup vs baseline: 1.5272x; 1.5272x over previous
import functools

import jax
import jax.numpy as jnp
from jax import lax
from jax.experimental import pallas as pl
from jax.experimental.pallas import tpu as pltpu
from jax.experimental.pallas import tpu_sc as plsc

N = 10000
E = 320000
D = 128
C = 40

NC = 2
NS = 16
NW = NC * NS

BLK = 64
EPW_BLKS = 160
HOLD = 40
NBUF = 4
E_PAD = NW * BLK * EPW_BLKS
TOT_BLKS = E_PAD // BLK
BLKS_C0 = 160
BLKS_C1 = 160
NROW = 10112
STRIPE = NROW // NS
DW = 128
ZCH = 128
DBLK = 128
DEG_BLKS = E_PAD // (NW * DBLK)

_mesh = plsc.VectorSubcoreMesh(
    core_axis_name="c", subcore_axis_name="s", num_cores=NC, num_subcores=NS)


def _zero_vmem_rows(ref, nrows, width):
  def row(r, _):
    for j in range(width // 16):
      ref[r, pl.ds(j * 16, 16)] = jnp.zeros((16,), jnp.float32)
    return 0
  lax.fori_loop(0, nrows, row, 0)


def _stripe_copy_zero(zbuf, zrows, acc, base, total):
  off = 0
  while off < total:
    ch = min(zrows, total - off)
    pltpu.sync_copy(zbuf.at[pl.ds(0, ch)], acc.at[pl.ds(base + off, ch)])
    off += ch


@functools.partial(
    pl.kernel,
    out_type=jax.ShapeDtypeStruct((NC, NROW, DW), jnp.float32),
    mesh=_mesh,
    scratch_types=[
        pltpu.VMEM((DEG_BLKS, DBLK), jnp.int32),
        pltpu.VMEM((ZCH, DW), jnp.float32),
        pltpu.VMEM((ZCH, DW), jnp.float32),
        pltpu.VMEM_SHARED((NROW, DW), jnp.float32),
    ],
)
def _sc_deg(dst_hbm, out_hbm, idst, ones_v, zbuf, acc):
  cid = lax.axis_index("c")
  sid = lax.axis_index("s")
  wid = sid * NC + cid

  def fill(r, _):
    for j in range(DW // 16):
      ones_v[r, pl.ds(j * 16, 16)] = jnp.ones((16,), jnp.float32)
      zbuf[r, pl.ds(j * 16, 16)] = jnp.zeros((16,), jnp.float32)
    return 0
  lax.fori_loop(0, ZCH, fill, 0)
  _stripe_copy_zero(zbuf, ZCH, acc, sid * STRIPE, STRIPE)
  pltpu.sync_copy(dst_hbm.at[wid], idst)
  plsc.subcore_barrier()

  def rnd(g, _):
    pltpu.sync_copy(ones_v.at[pl.ds(0, DBLK)], acc.at[idst.at[g]], add=True)
    return 0
  lax.fori_loop(0, DEG_BLKS, rnd, 0)
  plsc.subcore_barrier()
  pltpu.sync_copy(acc.at[pl.ds(sid * STRIPE, STRIPE)],
                  out_hbm.at[cid, pl.ds(sid * STRIPE, STRIPE)])


@functools.partial(
    pl.kernel,
    out_type=jax.ShapeDtypeStruct((NC, NROW, D), jnp.float32),
    mesh=_mesh,
    scratch_types=[
        pltpu.VMEM((HOLD, BLK), jnp.int32),
        pltpu.VMEM((HOLD, BLK), jnp.int32),
        [pltpu.VMEM((BLK, D // 2), jnp.int32)] * NBUF,
        pltpu.VMEM((BLK, D), jnp.float32),
        pltpu.VMEM_SHARED((NROW, D), jnp.float32),
        [pltpu.SemaphoreType.DMA] * NBUF,
    ],
    compiler_params=pltpu.CompilerParams(use_tc_tiling_on_sc=False),
)
def _sc_agg(hp_hbm, src_hbm, dst_hbm, out_hbm, isrc, idst, rows, rowf, acc,
            sems):
  cid = lax.axis_index("c")
  sid = lax.axis_index("s")

  _zero_vmem_rows(rowf, BLK, D)
  _stripe_copy_zero(rowf, BLK, acc, sid * STRIPE, STRIPE)
  plsc.subcore_barrier()

  base = jnp.where(cid == 0, sid * BLKS_C0, NS * BLKS_C0 + sid * BLKS_C1)
  nchunks = jnp.where(cid == 0, BLKS_C0 // HOLD, BLKS_C1 // HOLD)
  mask_hi = jnp.full((16,), -65536, jnp.int32)
  shift16 = jnp.full((16,), 16, jnp.int32)

  def chunk(q, _):
    b0 = base + q * HOLD
    pltpu.sync_copy(src_hbm.at[pl.ds(b0, HOLD)], isrc)
    pltpu.sync_copy(dst_hbm.at[pl.ds(b0, HOLD)], idst)
    for j in range(NBUF):
      pltpu.async_copy(hp_hbm.at[isrc.at[j]], rows[j], sems[j])

    def rnd(g, _):
      for j in range(NBUF):
        b = NBUF * g + j
        pltpu.make_async_copy(hp_hbm.at[isrc.at[b]], rows[j], sems[j]).wait()

        def conv(r, _):
          for g2 in range(D // 32):
            xi = rows[j][r, pl.ds(16 * g2, 16)]
            lo = lax.bitcast_convert_type(lax.shift_left(xi, shift16),
                                          jnp.float32)
            hi = lax.bitcast_convert_type(lax.bitwise_and(xi, mask_hi),
                                          jnp.float32)
            rowf[r, pl.ds(32 * g2, 16)] = lo
            rowf[r, pl.ds(32 * g2 + 16, 16)] = hi
          return 0
        lax.fori_loop(0, BLK, conv, 0)

        pltpu.sync_copy(rowf, acc.at[idst.at[b]], add=True)

        @pl.when(b + NBUF < HOLD)
        def _():
          pltpu.async_copy(hp_hbm.at[isrc.at[b + NBUF]], rows[j], sems[j])
      return 0

    lax.fori_loop(0, HOLD // NBUF, rnd, 0)
    return 0

  lax.fori_loop(0, nchunks, chunk, 0)
  plsc.subcore_barrier()
  pltpu.sync_copy(acc.at[pl.ds(sid * STRIPE, STRIPE)],
                  out_hbm.at[cid, pl.ds(sid * STRIPE, STRIPE)])


_RB = 1000


def _dinv_body(d0_ref, d1_ref, o_ref):
  deg = d0_ref[:, 0:1] + d1_ref[:, 0:1] + 1.0
  o_ref[...] = jnp.broadcast_to(lax.rsqrt(deg), o_ref.shape)


def _tc_dinv(deg_parts):
  return pl.pallas_call(
      _dinv_body,
      out_shape=jax.ShapeDtypeStruct((NROW, D), jnp.float32),
  )(deg_parts[0], deg_parts[1])


def _pack_rows(res, p_ref):
  perm = jnp.dot(res, p_ref[...], preferred_element_type=jnp.float32)
  bits = lax.bitcast_convert_type(
      perm.astype(jnp.bfloat16).astype(jnp.float32), jnp.int32)
  lo = lax.shift_right_logical(bits[:, :D // 2], 16)
  hi = lax.bitwise_and(bits[:, D // 2:], jnp.int32(-65536))
  return lax.bitwise_or(lo, hi)


def _mm_scale_body(x_ref, w_ref, dinv_ref, p_ref, o_ref, ob_ref):
  h = jnp.dot(x_ref[...], w_ref[...], preferred_element_type=jnp.float32)
  h = h * dinv_ref[...]
  o_ref[...] = h
  ob_ref[...] = _pack_rows(h, p_ref)


def _tc_mm_scale(x, w, dinv, pmat):
  grid = (N // _RB,)
  return pl.pallas_call(
      _mm_scale_body,
      grid=grid,
      in_specs=[
          pl.BlockSpec((_RB, D), lambda i: (i, 0)),
          pl.BlockSpec((D, D), lambda i: (0, 0)),
          pl.BlockSpec((_RB, D), lambda i: (i, 0)),
          pl.BlockSpec((D, D), lambda i: (0, 0)),
      ],
      out_specs=(pl.BlockSpec((_RB, D), lambda i: (i, 0)),
                 pl.BlockSpec((_RB, D // 2), lambda i: (i, 0))),
      out_shape=(jax.ShapeDtypeStruct((N, D), jnp.float32),
                 jax.ShapeDtypeStruct((N, D // 2), jnp.int32)),
  )(x, w, dinv, pmat)


def _scale_after_body(p0_ref, p1_ref, hp_ref, dinv_ref, b_ref, w_ref, p_ref,
                      o_ref, ob_ref):
  a = dinv_ref[...] * (p0_ref[...] + p1_ref[...] + hp_ref[...]) + b_ref[...]
  a = jnp.maximum(a, 0.0)
  res = jnp.dot(a, w_ref[...], preferred_element_type=jnp.float32)
  res = res * dinv_ref[...]
  o_ref[...] = res
  ob_ref[...] = _pack_rows(res, p_ref)


def _final_body(p0_ref, p1_ref, hp_ref, dinv_ref, b_ref, w_ref, bf_ref, o_ref):
  a = dinv_ref[...] * (p0_ref[...] + p1_ref[...] + hp_ref[...]) + b_ref[...]
  a = jnp.maximum(a, 0.0)
  o_ref[...] = jnp.dot(a, w_ref[...], preferred_element_type=jnp.float32) + bf_ref[...]


def _tc_combine(body, parts, hp, dinv, b_row, w, extra=(), out_shape=None,
                out_specs=None):
  grid = (N // _RB,)
  blk = pl.BlockSpec((_RB, D), lambda i: (i, 0))
  full = pl.BlockSpec((D, D), lambda i: (0, 0))
  brow = pl.BlockSpec((1, D), lambda i: (0, 0))
  in_specs = [blk, blk, blk, blk, brow, full] + [
      full if e.shape == (D, D) else brow for e in extra]
  if out_shape is None:
    out_shape = jax.ShapeDtypeStruct((N, D), jnp.float32)
    out_specs = blk
  return pl.pallas_call(
      body,
      grid=grid,
      in_specs=in_specs,
      out_specs=out_specs,
      out_shape=out_shape,
  )(parts[0], parts[1], hp, dinv, b_row, w, *extra)


def kernel(x, edge_index, W1, b1, Wh0, bh0, Wh1, bh1, Wf, bf):
  src = edge_index[0]
  dst = edge_index[1]
  pad = E_PAD - E
  src_p = jnp.concatenate([src, jnp.zeros((pad,), jnp.int32)])
  dst_p = jnp.concatenate([dst, jnp.full((pad,), N, jnp.int32)])
  src_r = src_p.reshape(TOT_BLKS, BLK)
  dst_r = dst_p.reshape(TOT_BLKS, BLK)

  pm = [[0.0] * D for _ in range(D)]
  for q in range(D // 32):
    for j in range(16):
      pm[32 * q + j][16 * q + j] = 1.0
      pm[32 * q + 16 + j][D // 2 + 16 * q + j] = 1.0
  pmat = jnp.asarray(pm, dtype=jnp.float32)

  deg_parts = _sc_deg(dst_p.reshape(NW, DEG_BLKS, DBLK))
  dinv_full = _tc_dinv(deg_parts)
  dinv = dinv_full[:N]

  wf_pad = jnp.zeros((D, D), jnp.float32).at[:, :C].set(Wf)
  bf_pad = jnp.zeros((1, D), jnp.float32).at[0, :C].set(bf)

  two_out = (jax.ShapeDtypeStruct((N, D), jnp.float32),
             jax.ShapeDtypeStruct((N, D // 2), jnp.int32))
  two_spec = (pl.BlockSpec((_RB, D), lambda i: (i, 0)),
              pl.BlockSpec((_RB, D // 2), lambda i: (i, 0)))

  hp, hp_b = _tc_mm_scale(x, W1, dinv, pmat)
  parts = _sc_agg(hp_b, src_r, dst_r)
  hp, hp_b = _tc_combine(
      _scale_after_body, (parts[0][:N], parts[1][:N]), hp, dinv,
      b1.reshape(1, D), Wh0, extra=(pmat,), out_shape=two_out,
      out_specs=two_spec)
  parts = _sc_agg(hp_b, src_r, dst_r)
  hp, hp_b = _tc_combine(
      _scale_after_body, (parts[0][:N], parts[1][:N]), hp, dinv,
      bh0.reshape(1, D), Wh1, extra=(pmat,), out_shape=two_out,
      out_specs=two_spec)
  parts = _sc_agg(hp_b, src_r, dst_r)
  out = _tc_combine(_final_body, (parts[0][:N], parts[1][:N]), hp, dinv,
                    bh1.reshape(1, D), wf_pad, extra=(bf_pad,))
  return out[:, :C]
